# Initial kernel scaffold; baseline (speedup 1.0000x reference)
import functools

import jax
import jax.numpy as jnp
from jax import lax
from jax.experimental import pallas as pl
from jax.experimental.pallas import tpu as pltpu
from jax.experimental.pallas import tpu_sc as plsc

N = 10000
E_RAW = 320000
D_IN = 128
D_H = 256
HALF = 128

CH = 128
E2P = 331776
PAD = E2P - (E_RAW + N)
CPR = E2P // CH
CPT = CPR // 16
CPW = CPR // 32

ROWS = 10016
STR = ROWS // 16
HSTR = STR // 2

DROWS = 10240
DSTR = DROWS // 16

NB = 1000
G = N // NB

_mesh = plsc.VectorSubcoreMesh(core_axis_name="c", subcore_axis_name="s")


@functools.partial(
    pl.kernel,
    out_type=jax.ShapeDtypeStruct((2, DROWS), jnp.float32),
    mesh=_mesh,
    scratch_types=[
        pltpu.VMEM((CPW, CH), jnp.int32),
        pltpu.VMEM((CH,), jnp.float32),
        pltpu.VMEM((DSTR,), jnp.float32),
        pltpu.VMEM_SHARED((DROWS,), jnp.float32),
    ],
)
def _sc_deg(dst_h, deg_out, dstb, ones, zbuf, accum):
    c = lax.axis_index("c")
    s = lax.axis_index("s")
    w = s * 2 + c

    def fill(i, _):
        zbuf[pl.ds(i * 16, 16)] = jnp.zeros((16,), jnp.float32)
        ones[pl.ds((i % 8) * 16, 16)] = jnp.ones((16,), jnp.float32)
        return 0

    lax.fori_loop(0, DSTR // 16, fill, 0)
    pltpu.sync_copy(zbuf, accum.at[pl.ds(s * DSTR, DSTR)])
    plsc.subcore_barrier()

    pltpu.sync_copy(dst_h.at[pl.ds(w * CPW, CPW)], dstb)

    def chunk(j, _):
        pltpu.sync_copy(ones, accum.at[dstb.at[j]], add=True)
        return 0

    lax.fori_loop(0, CPW, chunk, 0)
    plsc.subcore_barrier()
    pltpu.sync_copy(accum.at[pl.ds(s * DSTR, DSTR)], zbuf)
    pltpu.sync_copy(zbuf, deg_out.at[c, pl.ds(s * DSTR, DSTR)])


@functools.partial(
    pl.kernel,
    out_type=(jax.ShapeDtypeStruct((ROWS, HALF), jnp.float32),
              jax.ShapeDtypeStruct((ROWS, HALF), jnp.float32)),
    mesh=_mesh,
    scratch_types=[
        pltpu.VMEM((CPT, CH), jnp.int32),
        pltpu.VMEM((CPT, CH), jnp.int32),
        pltpu.VMEM((CH, HALF), jnp.float32),
        pltpu.VMEM((HSTR, HALF), jnp.float32),
        pltpu.VMEM_SHARED((ROWS, HALF), jnp.float32),
        pltpu.SemaphoreType.DMA,
    ],
)
def _sc_conv(hs_lo, hs_hi, src_h, dst_h, out_lo, out_hi,
             src2, dst2, rows, zbuf, accum, sem):
    c = lax.axis_index("c")
    s = lax.axis_index("s")

    def fill(i, _):
        zbuf[i // 8, pl.ds((i % 8) * 16, 16)] = jnp.zeros((16,), jnp.float32)
        return 0

    lax.fori_loop(0, HSTR * (HALF // 16), fill, 0)
    base = s * STR
    pltpu.sync_copy(zbuf, accum.at[pl.ds(base, HSTR)])
    pltpu.sync_copy(zbuf, accum.at[pl.ds(base + HSTR, HSTR)])
    plsc.subcore_barrier()

    pltpu.sync_copy(src_h.at[pl.ds(s * CPT, CPT)], src2)
    pltpu.sync_copy(dst_h.at[pl.ds(s * CPT, CPT)], dst2)

    def run(table, out):
        def chunk(j, _):
            pltpu.async_copy(table.at[src2.at[j]], rows, sem).wait()
            pltpu.sync_copy(rows, accum.at[dst2.at[j]], add=True)
            return 0

        lax.fori_loop(0, CPT, chunk, 0)
        plsc.subcore_barrier()
        pltpu.sync_copy(accum.at[pl.ds(base, HSTR)], zbuf)
        pltpu.sync_copy(zbuf, out.at[pl.ds(base, HSTR)])
        pltpu.sync_copy(accum.at[pl.ds(base + HSTR, HSTR)], zbuf)
        pltpu.sync_copy(zbuf, out.at[pl.ds(base + HSTR, HSTR)])

    @pl.when(c == 0)
    def _():
        run(hs_lo, out_lo)

    @pl.when(c == 1)
    def _():
        run(hs_hi, out_hi)


def _dense_in_body(x_ref, degp_ref, Win_ref, bin_ref, Wg1_ref,
                   h0_ref, hslo_ref, hshi_ref, dinv_ref):
    deg = degp_ref[:, 0:1] + degp_ref[:, 1:2]
    dinv = lax.rsqrt(deg)
    h0 = jnp.maximum(
        jnp.dot(x_ref[...], Win_ref[...], preferred_element_type=jnp.float32)
        + bin_ref[...], 0.0)
    hs = jnp.dot(h0, Wg1_ref[...], preferred_element_type=jnp.float32) * dinv
    h0_ref[...] = h0
    hslo_ref[...] = hs[:, :HALF]
    hshi_ref[...] = hs[:, HALF:]
    dinv_ref[...] = dinv


def _post_ln(agglo_ref, agghi_ref, dinv_ref, hid_ref, bg_ref, g_ref, be_ref):
    dinv = dinv_ref[...]
    agg = jnp.concatenate([agglo_ref[...], agghi_ref[...]], axis=1) * dinv
    agg = agg + bg_ref[...]
    mu = jnp.mean(agg, axis=1, keepdims=True)
    var = jnp.mean((agg - mu) ** 2, axis=1, keepdims=True)
    y = (agg - mu) * lax.rsqrt(var + 1e-5) * g_ref[...] + be_ref[...]
    return jnp.maximum(y, 0.0) + hid_ref[...], dinv


def _dense_mid_body(agglo_ref, agghi_ref, dinv_ref, hid_ref,
                    bg_ref, g_ref, be_ref, Wg_ref,
                    h_ref, hslo_ref, hshi_ref):
    h, dinv = _post_ln(agglo_ref, agghi_ref, dinv_ref, hid_ref,
                       bg_ref, g_ref, be_ref)
    h_ref[...] = h
    hs = jnp.dot(h, Wg_ref[...], preferred_element_type=jnp.float32) * dinv
    hslo_ref[...] = hs[:, :HALF]
    hshi_ref[...] = hs[:, HALF:]


def _dense_out_body(agglo_ref, agghi_ref, dinv_ref, hid_ref,
                    bg_ref, g_ref, be_ref,
                    Wf1_ref, bf1_ref, Wf2_ref, bf2_ref, out_ref):
    h, _ = _post_ln(agglo_ref, agghi_ref, dinv_ref, hid_ref,
                    bg_ref, g_ref, be_ref)
    t = jnp.maximum(
        jnp.dot(h, Wf1_ref[...], preferred_element_type=jnp.float32)
        + bf1_ref[...], 0.0)
    out_ref[...] = (jnp.dot(t, Wf2_ref[...], preferred_element_type=jnp.float32)
                    + bf2_ref[...])


def _row_spec(w):
    return pl.BlockSpec((NB, w), lambda g: (g, 0))


def _full_spec(h, w):
    return pl.BlockSpec((h, w), lambda g: (0, 0))


_dense_in = pl.pallas_call(
    _dense_in_body,
    grid=(G,),
    in_specs=[_row_spec(D_IN), _row_spec(2), _full_spec(D_IN, D_H),
              _full_spec(1, D_H), _full_spec(D_H, D_H)],
    out_specs=[_row_spec(D_H), _row_spec(HALF), _row_spec(HALF), _row_spec(1)],
    out_shape=[jax.ShapeDtypeStruct((N, D_H), jnp.float32),
               jax.ShapeDtypeStruct((N, HALF), jnp.float32),
               jax.ShapeDtypeStruct((N, HALF), jnp.float32),
               jax.ShapeDtypeStruct((N, 1), jnp.float32)],
)

_dense_mid = pl.pallas_call(
    _dense_mid_body,
    grid=(G,),
    in_specs=[_row_spec(HALF), _row_spec(HALF), _row_spec(1), _row_spec(D_H),
              _full_spec(1, D_H), _full_spec(1, D_H), _full_spec(1, D_H),
              _full_spec(D_H, D_H)],
    out_specs=[_row_spec(D_H), _row_spec(HALF), _row_spec(HALF)],
    out_shape=[jax.ShapeDtypeStruct((N, D_H), jnp.float32),
               jax.ShapeDtypeStruct((N, HALF), jnp.float32),
               jax.ShapeDtypeStruct((N, HALF), jnp.float32)],
)

_dense_out = pl.pallas_call(
    _dense_out_body,
    grid=(G,),
    in_specs=[_row_spec(HALF), _row_spec(HALF), _row_spec(1), _row_spec(D_H),
              _full_spec(1, D_H), _full_spec(1, D_H), _full_spec(1, D_H),
              _full_spec(D_H, HALF), _full_spec(1, HALF),
              _full_spec(HALF, 1), _full_spec(1, 1)],
    out_specs=_row_spec(1),
    out_shape=jax.ShapeDtypeStruct((N, 1), jnp.float32),
)


def kernel(x, edge_index, W_in, b_in, Wg1, bg1, g1, be1,
           Wg2, bg2, g2, be2, Wf1, bf1, Wf2, bf2):
    ei = edge_index.astype(jnp.int32)
    loop = jnp.arange(N, dtype=jnp.int32)
    src = jnp.concatenate([ei[0], loop, jnp.zeros((PAD,), jnp.int32)])
    dst = jnp.concatenate([ei[1], loop, jnp.full((PAD,), N, jnp.int32)])
    src = src.reshape(CPR, CH)
    dst = dst.reshape(CPR, CH)

    degp = _sc_deg(dst).T

    h0, hs1lo, hs1hi, dinv = _dense_in(
        x, degp, W_in, b_in.reshape(1, D_H), Wg1)
    a1lo, a1hi = _sc_conv(hs1lo, hs1hi, src, dst)
    h1, hs2lo, hs2hi = _dense_mid(
        a1lo, a1hi, dinv, h0, bg1.reshape(1, D_H), g1.reshape(1, D_H),
        be1.reshape(1, D_H), Wg2)
    a2lo, a2hi = _sc_conv(hs2lo, hs2hi, src, dst)
    out2 = _dense_out(
        a2lo, a2hi, dinv, h1, bg2.reshape(1, D_H), g2.reshape(1, D_H),
        be2.reshape(1, D_H), Wf1, bf1.reshape(1, HALF),
        Wf2, bf2.reshape(1, 1))
    return out2[:, 0]


# speedup vs baseline: 3.4415x; 3.4415x over previous
import functools

import jax
import jax.numpy as jnp
from jax import lax
from jax.experimental import pallas as pl
from jax.experimental.pallas import tpu as pltpu
from jax.experimental.pallas import tpu_sc as plsc

N = 10000
E_RAW = 320000
D_IN = 128
D_H = 256
HALF = 128

CH = 128
E2P = 360448
PAD = E2P - (E_RAW + N)
CPR = E2P // CH
CPT = CPR // 16
CPW = CPR // 32

ROWS = 10240
STR = ROWS // 16
HSTR = STR // 2

DROWS = 10240
DSTR = DROWS // 16

NB = 1000
G = N // NB

@functools.cache
def _get_sc_deg():
    return functools.partial(
        pl.kernel,
        out_type=jax.ShapeDtypeStruct((2, DROWS), jnp.float32),
        mesh=plsc.VectorSubcoreMesh(core_axis_name="c", subcore_axis_name="s"),
        scratch_types=[
            pltpu.VMEM((8, CH), jnp.int32),
            pltpu.VMEM((CH,), jnp.float32),
            pltpu.VMEM((DSTR,), jnp.float32),
            pltpu.VMEM_SHARED((DROWS,), jnp.float32),
        ],
    )(_sc_deg_body)


def _sc_deg_body(dst_h, deg_out, dstb, ones, zbuf, accum):
    c = lax.axis_index("c")
    s = lax.axis_index("s")
    w = s * 2 + c

    def fill(i, _):
        zbuf[pl.ds(i * 16, 16)] = jnp.zeros((16,), jnp.float32)
        ones[pl.ds((i % 8) * 16, 16)] = jnp.ones((16,), jnp.float32)
        return 0

    lax.fori_loop(0, DSTR // 16, fill, 0)
    pltpu.sync_copy(zbuf, accum.at[pl.ds(s * DSTR, DSTR)])
    plsc.subcore_barrier()

    def group(gi, _):
        pltpu.sync_copy(dst_h.at[pl.ds(w * CPW + gi * 8, 8)], dstb)

        def chunk(j, _):
            pltpu.sync_copy(ones, accum.at[dstb.at[j]], add=True)
            return 0

        lax.fori_loop(0, 8, chunk, 0)
        return 0

    lax.fori_loop(0, CPW // 8, group, 0)
    plsc.subcore_barrier()
    pltpu.sync_copy(accum.at[pl.ds(s * DSTR, DSTR)], zbuf)
    pltpu.sync_copy(zbuf, deg_out.at[c, pl.ds(s * DSTR, DSTR)])


@functools.cache
def _get_sc_conv():
    return functools.partial(
        pl.kernel,
        out_type=(jax.ShapeDtypeStruct((ROWS, HALF), jnp.float32),
                  jax.ShapeDtypeStruct((ROWS, HALF), jnp.float32)),
        mesh=plsc.VectorSubcoreMesh(core_axis_name="c", subcore_axis_name="s"),
        scratch_types=[
            pltpu.VMEM((8, CH), jnp.int32),
            pltpu.VMEM((8, CH), jnp.int32),
            pltpu.VMEM((CH, HALF), jnp.float32),
            pltpu.VMEM((CH, HALF), jnp.float32),
            pltpu.VMEM_SHARED((ROWS, HALF), jnp.float32),
            pltpu.SemaphoreType.DMA,
        ],
    )(_sc_conv_body)


def _sc_conv_body(hs_lo, hs_hi, src_h, dst_h, out_lo, out_hi,
                  srcb, dstb, rows, zbuf, accum, sem):
    c = lax.axis_index("c")
    s = lax.axis_index("s")

    def fill(i, _):
        zbuf[i // 8, pl.ds((i % 8) * 16, 16)] = jnp.zeros((16,), jnp.float32)
        return 0

    lax.fori_loop(0, CH * (HALF // 16), fill, 0)
    base = s * STR
    for p in range(STR // CH):
        pltpu.sync_copy(zbuf, accum.at[pl.ds(base + p * CH, CH)])
    plsc.subcore_barrier()

    def run(table, out):
        def group(gi, _):
            g0 = s * CPT + gi * 8
            pltpu.sync_copy(src_h.at[pl.ds(g0, 8)], srcb)
            pltpu.sync_copy(dst_h.at[pl.ds(g0, 8)], dstb)

            def chunk(j, _):
                pltpu.async_copy(table.at[srcb.at[j]], rows, sem).wait()
                pltpu.sync_copy(rows, accum.at[dstb.at[j]], add=True)
                return 0

            lax.fori_loop(0, 8, chunk, 0)
            return 0

        lax.fori_loop(0, CPT // 8, group, 0)
        plsc.subcore_barrier()
        for p in range(STR // CH):
            pltpu.sync_copy(accum.at[pl.ds(base + p * CH, CH)], zbuf)
            pltpu.sync_copy(zbuf, out.at[pl.ds(base + p * CH, CH)])

    @pl.when(c == 0)
    def _():
        run(hs_lo, out_lo)

    @pl.when(c == 1)
    def _():
        run(hs_hi, out_hi)


def _dense_in_body(x_ref, degp_ref, Win_ref, bin_ref, Wg1_ref,
                   h0_ref, hslo_ref, hshi_ref, dinv_ref):
    deg = degp_ref[:, 0:1] + degp_ref[:, 1:2]
    dinv = lax.rsqrt(deg)
    h0 = jnp.maximum(
        jnp.dot(x_ref[...], Win_ref[...], preferred_element_type=jnp.float32)
        + bin_ref[...], 0.0)
    hs = jnp.dot(h0, Wg1_ref[...], preferred_element_type=jnp.float32) * dinv
    h0_ref[...] = h0
    hslo_ref[...] = hs[:, :HALF]
    hshi_ref[...] = hs[:, HALF:]
    dinv_ref[...] = dinv


def _post_ln(agglo_ref, agghi_ref, dinv_ref, hid_ref, bg_ref, g_ref, be_ref):
    dinv = dinv_ref[...]
    agg = jnp.concatenate([agglo_ref[...], agghi_ref[...]], axis=1) * dinv
    agg = agg + bg_ref[...]
    mu = jnp.mean(agg, axis=1, keepdims=True)
    var = jnp.mean((agg - mu) ** 2, axis=1, keepdims=True)
    y = (agg - mu) * lax.rsqrt(var + 1e-5) * g_ref[...] + be_ref[...]
    return jnp.maximum(y, 0.0) + hid_ref[...], dinv


def _dense_mid_body(agglo_ref, agghi_ref, dinv_ref, hid_ref,
                    bg_ref, g_ref, be_ref, Wg_ref,
                    h_ref, hslo_ref, hshi_ref):
    h, dinv = _post_ln(agglo_ref, agghi_ref, dinv_ref, hid_ref,
                       bg_ref, g_ref, be_ref)
    h_ref[...] = h
    hs = jnp.dot(h, Wg_ref[...], preferred_element_type=jnp.float32) * dinv
    hslo_ref[...] = hs[:, :HALF]
    hshi_ref[...] = hs[:, HALF:]


def _dense_out_body(agglo_ref, agghi_ref, dinv_ref, hid_ref,
                    bg_ref, g_ref, be_ref,
                    Wf1_ref, bf1_ref, Wf2_ref, bf2_ref, out_ref):
    h, _ = _post_ln(agglo_ref, agghi_ref, dinv_ref, hid_ref,
                    bg_ref, g_ref, be_ref)
    t = jnp.maximum(
        jnp.dot(h, Wf1_ref[...], preferred_element_type=jnp.float32)
        + bf1_ref[...], 0.0)
    out_ref[...] = (jnp.dot(t, Wf2_ref[...], preferred_element_type=jnp.float32)
                    + bf2_ref[...])


def _row_spec(w):
    return pl.BlockSpec((NB, w), lambda g: (g, 0))


def _full_spec(h, w):
    return pl.BlockSpec((h, w), lambda g: (0, 0))


_dense_in = pl.pallas_call(
    _dense_in_body,
    grid=(G,),
    in_specs=[_row_spec(D_IN), _row_spec(2), _full_spec(D_IN, D_H),
              _full_spec(1, D_H), _full_spec(D_H, D_H)],
    out_specs=[_row_spec(D_H), _row_spec(HALF), _row_spec(HALF), _row_spec(1)],
    out_shape=[jax.ShapeDtypeStruct((N, D_H), jnp.float32),
               jax.ShapeDtypeStruct((N, HALF), jnp.float32),
               jax.ShapeDtypeStruct((N, HALF), jnp.float32),
               jax.ShapeDtypeStruct((N, 1), jnp.float32)],
)

_dense_mid = pl.pallas_call(
    _dense_mid_body,
    grid=(G,),
    in_specs=[_row_spec(HALF), _row_spec(HALF), _row_spec(1), _row_spec(D_H),
              _full_spec(1, D_H), _full_spec(1, D_H), _full_spec(1, D_H),
              _full_spec(D_H, D_H)],
    out_specs=[_row_spec(D_H), _row_spec(HALF), _row_spec(HALF)],
    out_shape=[jax.ShapeDtypeStruct((N, D_H), jnp.float32),
               jax.ShapeDtypeStruct((N, HALF), jnp.float32),
               jax.ShapeDtypeStruct((N, HALF), jnp.float32)],
)

_dense_out = pl.pallas_call(
    _dense_out_body,
    grid=(G,),
    in_specs=[_row_spec(HALF), _row_spec(HALF), _row_spec(1), _row_spec(D_H),
              _full_spec(1, D_H), _full_spec(1, D_H), _full_spec(1, D_H),
              _full_spec(D_H, HALF), _full_spec(1, HALF),
              _full_spec(HALF, 1), _full_spec(1, 1)],
    out_specs=_row_spec(1),
    out_shape=jax.ShapeDtypeStruct((N, 1), jnp.float32),
)


def kernel(x, edge_index, W_in, b_in, Wg1, bg1, g1, be1,
           Wg2, bg2, g2, be2, Wf1, bf1, Wf2, bf2):
    ei = edge_index.astype(jnp.int32)
    loop = jnp.arange(N, dtype=jnp.int32)
    src = jnp.concatenate([ei[0], loop, jnp.zeros((PAD,), jnp.int32)])
    dst = jnp.concatenate([ei[1], loop, jnp.full((PAD,), N, jnp.int32)])
    src = src.reshape(CPR, CH)
    dst = dst.reshape(CPR, CH)

    degp = _get_sc_deg()(dst).T

    h0, hs1lo, hs1hi, dinv = _dense_in(
        x, degp, W_in, b_in.reshape(1, D_H), Wg1)
    a1lo, a1hi = _get_sc_conv()(hs1lo, hs1hi, src, dst)
    h1, hs2lo, hs2hi = _dense_mid(
        a1lo, a1hi, dinv, h0, bg1.reshape(1, D_H), g1.reshape(1, D_H),
        be1.reshape(1, D_H), Wg2)
    a2lo, a2hi = _get_sc_conv()(hs2lo, hs2hi, src, dst)
    out2 = _dense_out(
        a2lo, a2hi, dinv, h1, bg2.reshape(1, D_H), g2.reshape(1, D_H),
        be2.reshape(1, D_H), Wf1, bf1.reshape(1, HALF),
        Wf2, bf2.reshape(1, 1))
    return out2[:, 0]


# speedup vs baseline: 3.5429x; 1.0295x over previous
import functools

import jax
import jax.numpy as jnp
from jax import lax
from jax.experimental import pallas as pl
from jax.experimental.pallas import tpu as pltpu
from jax.experimental.pallas import tpu_sc as plsc

N = 10000
E_RAW = 320000
D_IN = 128
D_H = 256
HALF = 128

CH = 128
E2P = 360448
PAD = E2P - (E_RAW + N)
CPR = E2P // CH
CPT = CPR // 16
CPW = CPR // 32

ROWS = 10240
STR = ROWS // 16
HSTR = STR // 2

DROWS = 10240
DSTR = DROWS // 16

NB = 1000
G = N // NB

@functools.cache
def _get_sc_deg():
    return functools.partial(
        pl.kernel,
        out_type=jax.ShapeDtypeStruct((2, DROWS), jnp.float32),
        mesh=plsc.VectorSubcoreMesh(core_axis_name="c", subcore_axis_name="s"),
        scratch_types=[
            pltpu.VMEM((8, CH), jnp.int32),
            pltpu.VMEM((CH,), jnp.float32),
            pltpu.VMEM((DSTR,), jnp.float32),
            pltpu.VMEM_SHARED((DROWS,), jnp.float32),
        ],
    )(_sc_deg_body)


def _sc_deg_body(dst_h, deg_out, dstb, ones, zbuf, accum):
    c = lax.axis_index("c")
    s = lax.axis_index("s")
    w = s * 2 + c

    def fill(i, _):
        zbuf[pl.ds(i * 16, 16)] = jnp.zeros((16,), jnp.float32)
        ones[pl.ds((i % 8) * 16, 16)] = jnp.ones((16,), jnp.float32)
        return 0

    lax.fori_loop(0, DSTR // 16, fill, 0)
    pltpu.sync_copy(zbuf, accum.at[pl.ds(s * DSTR, DSTR)])
    plsc.subcore_barrier()

    def group(gi, _):
        pltpu.sync_copy(dst_h.at[pl.ds(w * CPW + gi * 8, 8)], dstb)

        def chunk(j, _):
            pltpu.sync_copy(ones, accum.at[dstb.at[j]], add=True)
            return 0

        lax.fori_loop(0, 8, chunk, 0)
        return 0

    lax.fori_loop(0, CPW // 8, group, 0)
    plsc.subcore_barrier()
    pltpu.sync_copy(accum.at[pl.ds(s * DSTR, DSTR)], zbuf)
    pltpu.sync_copy(zbuf, deg_out.at[c, pl.ds(s * DSTR, DSTR)])


@functools.cache
def _get_sc_conv():
    return functools.partial(
        pl.kernel,
        out_type=(jax.ShapeDtypeStruct((ROWS, HALF), jnp.float32),
                  jax.ShapeDtypeStruct((ROWS, HALF), jnp.float32)),
        mesh=plsc.VectorSubcoreMesh(core_axis_name="c", subcore_axis_name="s"),
        scratch_types=[
            pltpu.VMEM((8, CH), jnp.int32),
            pltpu.VMEM((8, CH), jnp.int32),
            pltpu.VMEM((CH, HALF), jnp.float32),
            pltpu.VMEM((CH, HALF), jnp.float32),
            pltpu.VMEM((64, HALF), jnp.float32),
            pltpu.VMEM_SHARED((ROWS, HALF), jnp.float32),
            pltpu.SemaphoreType.DMA,
            pltpu.SemaphoreType.DMA,
            pltpu.SemaphoreType.DMA,
            pltpu.SemaphoreType.DMA,
        ],
    )(_sc_conv_body)


def _sc_conv_body(hs_lo, hs_hi, src_h, dst_h, out_lo, out_hi,
                  srcb, dstb, rows0, rows1, zbuf, accum,
                  gsem0, gsem1, ssem0, ssem1):
    c = lax.axis_index("c")
    s = lax.axis_index("s")
    rows = (rows0, rows1)
    gsem = (gsem0, gsem1)
    ssem = (ssem0, ssem1)

    def fill(i, _):
        zbuf[i // 8, pl.ds((i % 8) * 16, 16)] = jnp.zeros((16,), jnp.float32)
        return 0

    lax.fori_loop(0, 64 * (HALF // 16), fill, 0)
    base = s * STR
    for p in range(STR // 64):
        pltpu.sync_copy(zbuf, accum.at[pl.ds(base + p * 64, 64)])
    plsc.subcore_barrier()

    def run(table, out):
        def gwait(b):
            pltpu.make_async_copy(table.at[pl.ds(0, CH)], rows[b],
                                  gsem[b]).wait()

        def swait(b):
            pltpu.make_async_copy(rows[b], out.at[pl.ds(0, CH)],
                                  ssem[b]).wait()

        def group(gi, _):
            g0 = s * CPT + gi * 8
            pltpu.sync_copy(src_h.at[pl.ds(g0, 8)], srcb)
            pltpu.sync_copy(dst_h.at[pl.ds(g0, 8)], dstb)
            pltpu.async_copy(table.at[srcb.at[0]], rows[0], gsem[0])
            for j in range(8):
                b = j % 2
                nb = 1 - b
                if j < 7:
                    if j >= 1:
                        swait(nb)
                    pltpu.async_copy(table.at[srcb.at[j + 1]], rows[nb],
                                     gsem[nb])
                gwait(b)
                pltpu.async_copy(rows[b], accum.at[dstb.at[j]], ssem[b],
                                 add=True)
            swait(0)
            swait(1)
            return 0

        lax.fori_loop(0, CPT // 8, group, 0)
        plsc.subcore_barrier()
        for p in range(STR // 64):
            pltpu.sync_copy(accum.at[pl.ds(base + p * 64, 64)], zbuf)
            pltpu.sync_copy(zbuf, out.at[pl.ds(base + p * 64, 64)])

    @pl.when(c == 0)
    def _():
        run(hs_lo, out_lo)

    @pl.when(c == 1)
    def _():
        run(hs_hi, out_hi)


def _dense_in_body(x_ref, degp_ref, Win_ref, bin_ref, Wg1_ref,
                   h0_ref, hslo_ref, hshi_ref, dinv_ref):
    deg = degp_ref[:, 0:1] + degp_ref[:, 1:2]
    dinv = lax.rsqrt(deg)
    h0 = jnp.maximum(
        jnp.dot(x_ref[...], Win_ref[...], preferred_element_type=jnp.float32)
        + bin_ref[...], 0.0)
    hs = jnp.dot(h0, Wg1_ref[...], preferred_element_type=jnp.float32) * dinv
    h0_ref[...] = h0
    hslo_ref[...] = hs[:, :HALF]
    hshi_ref[...] = hs[:, HALF:]
    dinv_ref[...] = dinv


def _post_ln(agglo_ref, agghi_ref, dinv_ref, hid_ref, bg_ref, g_ref, be_ref):
    dinv = dinv_ref[...]
    agg = jnp.concatenate([agglo_ref[...], agghi_ref[...]], axis=1) * dinv
    agg = agg + bg_ref[...]
    mu = jnp.mean(agg, axis=1, keepdims=True)
    var = jnp.mean((agg - mu) ** 2, axis=1, keepdims=True)
    y = (agg - mu) * lax.rsqrt(var + 1e-5) * g_ref[...] + be_ref[...]
    return jnp.maximum(y, 0.0) + hid_ref[...], dinv


def _dense_mid_body(agglo_ref, agghi_ref, dinv_ref, hid_ref,
                    bg_ref, g_ref, be_ref, Wg_ref,
                    h_ref, hslo_ref, hshi_ref):
    h, dinv = _post_ln(agglo_ref, agghi_ref, dinv_ref, hid_ref,
                       bg_ref, g_ref, be_ref)
    h_ref[...] = h
    hs = jnp.dot(h, Wg_ref[...], preferred_element_type=jnp.float32) * dinv
    hslo_ref[...] = hs[:, :HALF]
    hshi_ref[...] = hs[:, HALF:]


def _dense_out_body(agglo_ref, agghi_ref, dinv_ref, hid_ref,
                    bg_ref, g_ref, be_ref,
                    Wf1_ref, bf1_ref, Wf2_ref, bf2_ref, out_ref):
    h, _ = _post_ln(agglo_ref, agghi_ref, dinv_ref, hid_ref,
                    bg_ref, g_ref, be_ref)
    t = jnp.maximum(
        jnp.dot(h, Wf1_ref[...], preferred_element_type=jnp.float32)
        + bf1_ref[...], 0.0)
    out_ref[...] = (jnp.dot(t, Wf2_ref[...], preferred_element_type=jnp.float32)
                    + bf2_ref[...])


def _row_spec(w):
    return pl.BlockSpec((NB, w), lambda g: (g, 0))


def _full_spec(h, w):
    return pl.BlockSpec((h, w), lambda g: (0, 0))


_dense_in = pl.pallas_call(
    _dense_in_body,
    grid=(G,),
    in_specs=[_row_spec(D_IN), _row_spec(2), _full_spec(D_IN, D_H),
              _full_spec(1, D_H), _full_spec(D_H, D_H)],
    out_specs=[_row_spec(D_H), _row_spec(HALF), _row_spec(HALF), _row_spec(1)],
    out_shape=[jax.ShapeDtypeStruct((N, D_H), jnp.float32),
               jax.ShapeDtypeStruct((N, HALF), jnp.float32),
               jax.ShapeDtypeStruct((N, HALF), jnp.float32),
               jax.ShapeDtypeStruct((N, 1), jnp.float32)],
)

_dense_mid = pl.pallas_call(
    _dense_mid_body,
    grid=(G,),
    in_specs=[_row_spec(HALF), _row_spec(HALF), _row_spec(1), _row_spec(D_H),
              _full_spec(1, D_H), _full_spec(1, D_H), _full_spec(1, D_H),
              _full_spec(D_H, D_H)],
    out_specs=[_row_spec(D_H), _row_spec(HALF), _row_spec(HALF)],
    out_shape=[jax.ShapeDtypeStruct((N, D_H), jnp.float32),
               jax.ShapeDtypeStruct((N, HALF), jnp.float32),
               jax.ShapeDtypeStruct((N, HALF), jnp.float32)],
)

_dense_out = pl.pallas_call(
    _dense_out_body,
    grid=(G,),
    in_specs=[_row_spec(HALF), _row_spec(HALF), _row_spec(1), _row_spec(D_H),
              _full_spec(1, D_H), _full_spec(1, D_H), _full_spec(1, D_H),
              _full_spec(D_H, HALF), _full_spec(1, HALF),
              _full_spec(HALF, 1), _full_spec(1, 1)],
    out_specs=_row_spec(1),
    out_shape=jax.ShapeDtypeStruct((N, 1), jnp.float32),
)


def kernel(x, edge_index, W_in, b_in, Wg1, bg1, g1, be1,
           Wg2, bg2, g2, be2, Wf1, bf1, Wf2, bf2):
    ei = edge_index.astype(jnp.int32)
    loop = jnp.arange(N, dtype=jnp.int32)
    src = jnp.concatenate([ei[0], loop, jnp.zeros((PAD,), jnp.int32)])
    dst = jnp.concatenate([ei[1], loop, jnp.full((PAD,), N, jnp.int32)])
    src = src.reshape(CPR, CH)
    dst = dst.reshape(CPR, CH)

    degp = _get_sc_deg()(dst).T

    h0, hs1lo, hs1hi, dinv = _dense_in(
        x, degp, W_in, b_in.reshape(1, D_H), Wg1)
    a1lo, a1hi = _get_sc_conv()(hs1lo, hs1hi, src, dst)
    h1, hs2lo, hs2hi = _dense_mid(
        a1lo, a1hi, dinv, h0, bg1.reshape(1, D_H), g1.reshape(1, D_H),
        be1.reshape(1, D_H), Wg2)
    a2lo, a2hi = _get_sc_conv()(hs2lo, hs2hi, src, dst)
    out2 = _dense_out(
        a2lo, a2hi, dinv, h1, bg2.reshape(1, D_H), g2.reshape(1, D_H),
        be2.reshape(1, D_H), Wf1, bf1.reshape(1, HALF),
        Wf2, bf2.reshape(1, 1))
    return out2[:, 0]
